# baseline (speedup 1.0000x reference)
import numpy as np
import jax
import jax.numpy as jnp
from jax.experimental import pallas as pl
from jax.experimental.pallas import tpu as pltpu

_N, _L, _K = 128, 8192, 20
_BN = 8
_BL = 2048


def _threefry_bits(x1):
    k1 = jnp.uint32(42)
    k2 = jnp.uint32(0 ^ 42 ^ 0x1BD11BDA)
    ks = (jnp.uint32(0), k1, k2)
    rot = ((13, 15, 26, 6), (17, 29, 16, 24))
    x0 = x1
    x1 = ((x1 << 13) | (x1 >> 19)) ^ x0
    for i in range(5):
        rs = rot[i % 2][1:] if i == 0 else rot[i % 2]
        for r in rs:
            x0 = x0 + x1
            x1 = ((x1 << r) | (x1 >> (32 - r))) ^ x0
        x0 = x0 + ks[(i + 1) % 3]
        x1 = x1 + ks[(i + 2) % 3] + jnp.uint32(i + 1)
    return x0 ^ x1


_CH = 256


def _body(dt_ref, ct_ref, vc_ref, xt_ref, mk_ref, x_out, c_out):
    bn = pl.program_id(0)
    bl = pl.program_id(1)
    dtb = dt_ref[:, 0:1][None]
    tiny = jnp.float32(np.finfo(np.float32).tiny)

    row = jax.lax.broadcasted_iota(jnp.int32, (_K, _BN, _CH), 1)
    lane = jax.lax.broadcasted_iota(jnp.int32, (_K, _BN, _CH), 2)
    kpl = jax.lax.broadcasted_iota(jnp.int32, (_K, _BN, _CH), 0)
    tok = (bn * _BN + row) * _L + (bl * _BL + lane)
    x1n = (tok * _K + (kpl + 42)).astype(jnp.uint32)

    for c in range(_BL // _CH):
        sl = slice(c * _CH, (c + 1) * _CH)
        ct = ct_ref[:, :, sl]
        vc = vc_ref[:, :, sl]
        c_new = ct + vc * dtb
        probs = jnp.clip(c_new, 0.0, 1.0) + 1e-8
        v = jnp.log(probs)

        bits = _threefry_bits(x1n)
        x1n = x1n + jnp.uint32(_K * _CH)
        fb = (bits >> 9) | jnp.uint32(0x3F800000)
        floats = jax.lax.bitcast_convert_type(fb, jnp.float32) - 1.0
        u = jnp.maximum(tiny, floats + tiny)
        v = v + (-jnp.log(-jnp.log(u)))

        cur_v = v[0]
        cur_i = jnp.zeros((_BN, _CH), jnp.int32)
        for k in range(1, _K):
            vk = v[k]
            gt = vk > cur_v
            cur_v = jnp.where(gt, vk, cur_v)
            cur_i = jnp.where(gt, k, cur_i)

        mk = mk_ref[:, sl]
        x_out[:, sl] = jnp.where(mk != 0, cur_i, xt_ref[:, sl])
        c_out[:, :, sl] = jnp.where(mk[None] != 0, c_new, ct)


def kernel(xt, ct, vc_t, dt, mask):
    ctT = jnp.transpose(ct, (2, 0, 1))
    vcT = jnp.transpose(vc_t, (2, 0, 1))
    mk = mask.astype(jnp.int32)
    dtl = jnp.broadcast_to(dt[:, None], (_N, 128))
    x_new, c_newT = pl.pallas_call(
        _body,
        grid=(_N // _BN, _L // _BL),
        in_specs=[
            pl.BlockSpec((_BN, 128), lambda bn, bl: (bn, 0)),
            pl.BlockSpec((_K, _BN, _BL), lambda bn, bl: (0, bn, bl)),
            pl.BlockSpec((_K, _BN, _BL), lambda bn, bl: (0, bn, bl)),
            pl.BlockSpec((_BN, _BL), lambda bn, bl: (bn, bl)),
            pl.BlockSpec((_BN, _BL), lambda bn, bl: (bn, bl)),
        ],
        out_specs=[
            pl.BlockSpec((_BN, _BL), lambda bn, bl: (bn, bl)),
            pl.BlockSpec((_K, _BN, _BL), lambda bn, bl: (0, bn, bl)),
        ],
        out_shape=[
            jax.ShapeDtypeStruct((_N, _L), jnp.int32),
            jax.ShapeDtypeStruct((_K, _N, _L), jnp.float32),
        ],
        compiler_params=pltpu.CompilerParams(
            dimension_semantics=("parallel", "parallel")),
    )(dtl, ctT, vcT, xt, mk)
    return x_new, jnp.transpose(c_newT, (1, 2, 0))


# speedup vs baseline: 1.1382x; 1.1382x over previous
import numpy as np
import jax
import jax.numpy as jnp
from jax.experimental import pallas as pl
from jax.experimental.pallas import tpu as pltpu

_N, _L, _K = 128, 8192, 20
_BN = 8
_BL = 2048


def _threefry_bits(x1):
    k1 = jnp.uint32(42)
    k2 = jnp.uint32(0 ^ 42 ^ 0x1BD11BDA)
    ks = (jnp.uint32(0), k1, k2)
    rot = ((13, 15, 26, 6), (17, 29, 16, 24))
    x0 = x1
    x1 = ((x1 << 13) | (x1 >> 19)) ^ x0
    for i in range(5):
        rs = rot[i % 2][1:] if i == 0 else rot[i % 2]
        for r in rs:
            x0 = x0 + x1
            x1 = ((x1 << r) | (x1 >> (32 - r))) ^ x0
        x0 = x0 + ks[(i + 1) % 3]
        x1 = x1 + ks[(i + 2) % 3] + jnp.uint32(i + 1)
    return x0 ^ x1


_CH = 128


def _body(dt_ref, ct_ref, vc_ref, xt_ref, mk_ref, x_out, c_out):
    bn = pl.program_id(0)
    bl = pl.program_id(1)
    dtb = dt_ref[:, 0:1][None]
    tiny = jnp.float32(np.finfo(np.float32).tiny)

    row = jax.lax.broadcasted_iota(jnp.int32, (_K, _BN, _CH), 1)
    lane = jax.lax.broadcasted_iota(jnp.int32, (_K, _BN, _CH), 2)
    kpl = jax.lax.broadcasted_iota(jnp.int32, (_K, _BN, _CH), 0)
    tok = (bn * _BN + row) * _L + (bl * _BL + lane)
    x1n = (tok * _K + (kpl + 42)).astype(jnp.uint32)

    for c in range(_BL // _CH):
        sl = slice(c * _CH, (c + 1) * _CH)
        ct = ct_ref[:, :, sl]
        vc = vc_ref[:, :, sl]
        c_new = ct + vc * dtb
        probs = jnp.clip(c_new, 0.0, 1.0) + 1e-8
        v = jnp.log(probs)

        bits = _threefry_bits(x1n)
        x1n = x1n + jnp.uint32(_K * _CH)
        fb = (bits >> 9) | jnp.uint32(0x3F800000)
        floats = jax.lax.bitcast_convert_type(fb, jnp.float32) - 1.0
        u = jnp.maximum(tiny, floats)
        v = v - jnp.log(-jnp.log(u))

        cur_v = v[0]
        cur_i = jnp.zeros((_BN, _CH), jnp.int32)
        for k in range(1, _K):
            vk = v[k]
            gt = vk > cur_v
            cur_v = jnp.where(gt, vk, cur_v)
            cur_i = jnp.where(gt, k, cur_i)

        mk = mk_ref[:, sl]
        x_out[:, sl] = jnp.where(mk != 0, cur_i, xt_ref[:, sl])
        c_out[:, :, sl] = jnp.where(mk[None] != 0, c_new, ct)


def kernel(xt, ct, vc_t, dt, mask):
    ctT = jnp.transpose(ct, (2, 0, 1))
    vcT = jnp.transpose(vc_t, (2, 0, 1))
    mk = mask.astype(jnp.int32)
    dtl = jnp.broadcast_to(dt[:, None], (_N, 128))
    x_new, c_newT = pl.pallas_call(
        _body,
        grid=(_N // _BN, _L // _BL),
        in_specs=[
            pl.BlockSpec((_BN, 128), lambda bn, bl: (bn, 0)),
            pl.BlockSpec((_K, _BN, _BL), lambda bn, bl: (0, bn, bl)),
            pl.BlockSpec((_K, _BN, _BL), lambda bn, bl: (0, bn, bl)),
            pl.BlockSpec((_BN, _BL), lambda bn, bl: (bn, bl)),
            pl.BlockSpec((_BN, _BL), lambda bn, bl: (bn, bl)),
        ],
        out_specs=[
            pl.BlockSpec((_BN, _BL), lambda bn, bl: (bn, bl)),
            pl.BlockSpec((_K, _BN, _BL), lambda bn, bl: (0, bn, bl)),
        ],
        out_shape=[
            jax.ShapeDtypeStruct((_N, _L), jnp.int32),
            jax.ShapeDtypeStruct((_K, _N, _L), jnp.float32),
        ],
        compiler_params=pltpu.CompilerParams(
            dimension_semantics=("parallel", "parallel")),
    )(dtl, ctT, vcT, xt, mk)
    return x_new, jnp.transpose(c_newT, (1, 2, 0))
